# Initial kernel scaffold; baseline (speedup 1.0000x reference)
import jax
import jax.numpy as jnp
from jax.experimental import pallas as pl


def kernel(x, edge_attr, params, edge_index):
    raise NotImplementedError("write your pallas kernel here")


# speedup vs baseline: 19.3685x; 19.3685x over previous
import functools

import numpy as np
import jax
import jax.numpy as jnp
from jax import lax
from jax.experimental import pallas as pl
from jax.experimental.pallas import tpu as pltpu
from jax.experimental.pallas import tpu_sc as plsc

N = 10000
E = 320000
HID = 128
HEADS = 8
DH = 16
LAYERS = 4

f32 = jnp.float32

NC = 2
NS = 16
NW = NC * NS
EW = E // NW
C = 40
G = EW // C
NP = 10240
RPT = NP // NS

_DENB_NP = np.zeros((HEADS, HID), np.float32)
for _h in range(HEADS):
    _DENB_NP[_h, _h * DH:(_h + 1) * DH] = 1.0


def _mm(a, b):
    return jnp.dot(a, b, preferred_element_type=f32)


def _bn(v, g, b):
    m = jnp.mean(v, axis=0, keepdims=True)
    var = jnp.mean((v - m) ** 2, axis=0, keepdims=True)
    return (v - m) * lax.rsqrt(var + 1e-5) * g + b


@functools.lru_cache(maxsize=None)
def _build_sc_a():
    mesh = plsc.VectorSubcoreMesh(core_axis_name="c", subcore_axis_name="s")

    def body(q_hbm, k_hbm, ep_hbm, src_hbm, dst_hbm, eout_hbm,
             srcv, dstv, qb, kb, epb, sb, sem):
        cid = lax.axis_index("c")
        sid = lax.axis_index("s")
        wid = sid * NC + cid
        base_w = wid * EW

        def chunk(g, carry):
            base = pl.multiple_of(base_w + g * C, 8)
            pltpu.sync_copy(src_hbm.at[pl.ds(base, C)], srcv)
            pltpu.sync_copy(dst_hbm.at[pl.ds(base, C)], dstv)
            cps = (pltpu.async_copy(q_hbm.at[dstv], qb, sem),
                   pltpu.async_copy(k_hbm.at[srcv], kb, sem),
                   pltpu.async_copy(ep_hbm.at[pl.ds(base, C)], epb, sem))
            for cp in cps:
                cp.wait()

            def edge(i, c2):
                for h in range(HEADS):
                    sl = pl.ds(h * DH, DH)
                    sb[i, sl] = qb[i, sl] * kb[i, sl] * 0.25 * epb[i, sl]
                return c2

            lax.fori_loop(0, C, edge, 0)
            pltpu.sync_copy(sb, eout_hbm.at[pl.ds(base, C)])
            return carry

        lax.fori_loop(0, G, chunk, 0)

    return pl.kernel(
        body,
        out_type=jax.ShapeDtypeStruct((E, HID), f32),
        mesh=mesh,
        scratch_types=[
            pltpu.VMEM((C,), jnp.int32),
            pltpu.VMEM((C,), jnp.int32),
            pltpu.VMEM((C, HID), f32),
            pltpu.VMEM((C, HID), f32),
            pltpu.VMEM((C, HID), f32),
            pltpu.VMEM((C, HID), f32),
            pltpu.SemaphoreType.DMA,
        ],
    )


@functools.lru_cache(maxsize=None)
def _build_sc_b():
    mesh = plsc.VectorSubcoreMesh(core_axis_name="c", subcore_axis_name="s")

    def body(v_hbm, wb_hbm, src_hbm, dst_hbm, ndn_hbm, ndd_hbm,
             srcv, dstv, vb, wbb, ndb, acc, sem):
        cid = lax.axis_index("c")
        sid = lax.axis_index("s")
        wid = sid * NC + cid
        base_w = wid * EW
        z16 = jnp.zeros((16,), f32)

        def zero_ndb():
            def zrow(r, carry):
                for j in range(HID // 16):
                    ndb[r, pl.ds(j * 16, 16)] = z16
                return carry

            lax.fori_loop(0, C, zrow, 0)

        def zero_acc():
            for j in range(RPT // C):
                pltpu.sync_copy(ndb, acc.at[pl.ds(sid * RPT + j * C, C)])

        zero_ndb()
        zero_acc()
        plsc.subcore_barrier()

        def chunk1(g, carry):
            base = pl.multiple_of(base_w + g * C, 8)
            pltpu.sync_copy(src_hbm.at[pl.ds(base, C)], srcv)
            pltpu.sync_copy(dst_hbm.at[pl.ds(base, C)], dstv)
            cps = (pltpu.async_copy(v_hbm.at[srcv], vb, sem),
                   pltpu.async_copy(wb_hbm.at[pl.ds(base, C)], wbb, sem))
            for cp in cps:
                cp.wait()

            def edge(i, c2):
                for h in range(HEADS):
                    sl = pl.ds(h * DH, DH)
                    ndb[i, sl] = vb[i, sl] * wbb[i, sl]
                return c2

            lax.fori_loop(0, C, edge, 0)
            pltpu.sync_copy(ndb, acc.at[dstv], add=True)
            return carry

        lax.fori_loop(0, G, chunk1, 0)
        plsc.subcore_barrier()
        pltpu.sync_copy(acc.at[pl.ds(sid * RPT, RPT)],
                        ndn_hbm.at[cid, pl.ds(sid * RPT, RPT)])
        zero_ndb()
        zero_acc()
        plsc.subcore_barrier()

        def chunk2(g, carry):
            base = pl.multiple_of(base_w + g * C, 8)
            pltpu.sync_copy(dst_hbm.at[pl.ds(base, C)], dstv)
            pltpu.async_copy(wb_hbm.at[pl.ds(base, C)], wbb, sem).wait()
            pltpu.sync_copy(wbb, acc.at[dstv], add=True)
            return carry

        lax.fori_loop(0, G, chunk2, 0)
        plsc.subcore_barrier()
        pltpu.sync_copy(acc.at[pl.ds(sid * RPT, RPT)],
                        ndd_hbm.at[cid, pl.ds(sid * RPT, RPT)])

    return pl.kernel(
        body,
        out_type=(jax.ShapeDtypeStruct((NC, NP, HID), f32),
                  jax.ShapeDtypeStruct((NC, NP, HID), f32)),
        mesh=mesh,
        scratch_types=[
            pltpu.VMEM((C,), jnp.int32),
            pltpu.VMEM((C,), jnp.int32),
            pltpu.VMEM((C, HID), f32),
            pltpu.VMEM((C, HID), f32),
            pltpu.VMEM((C, HID), f32),
            pltpu.VMEM_SHARED((NP, HID), f32),
            pltpu.SemaphoreType.DMA,
        ],
    )


def _node0_body(x, ne, wq, wk, wv, h_o, q_o, k_o, v_o):
    h = _mm(x[...], ne[...])
    h_o[...] = h
    q_o[...] = _mm(h, wq[...])
    k_o[...] = _mm(h, wk[...])
    v_o[...] = _mm(h, wv[...])


def _build_node0(interpret=False):
    return pl.pallas_call(
        _node0_body,
        out_shape=(jax.ShapeDtypeStruct((N, HID), f32),) * 4,
        interpret=interpret,
    )


def _attn_h2(h, ndn, ndd, wo, g1, b1, w1, bb1, w2, bb2, g2, b2):
    num = ndn[0, :N] + ndn[1, :N]
    den = ndd[0, :N] + ndd[1, :N]
    attn = num / (den + 1e-6)
    t = h + _mm(attn, wo)
    h1 = _bn(t, g1, b1)
    ff = _mm(jax.nn.relu(_mm(h1, w1) + bb1), w2) + bb2
    return _bn(h1 + ff, g2, b2)


def _node_mid_body(h, ndn, ndd, wo, g1, b1, w1, bb1, w2, bb2, g2, b2,
                   wq, wk, wv, h_o, q_o, k_o, v_o):
    h2 = _attn_h2(h[...], ndn[...], ndd[...], wo[...], g1[...], b1[...],
                  w1[...], bb1[...], w2[...], bb2[...], g2[...], b2[...])
    h_o[...] = h2
    q_o[...] = _mm(h2, wq[...])
    k_o[...] = _mm(h2, wk[...])
    v_o[...] = _mm(h2, wv[...])


def _build_node_mid(interpret=False):
    return pl.pallas_call(
        _node_mid_body,
        out_shape=(jax.ShapeDtypeStruct((N, HID), f32),) * 4,
        interpret=interpret,
    )


def _node_final_body(h, ndn, ndd, wo, g1, b1, w1, bb1, w2, bb2, g2, b2,
                     r0w, r0b, r1w, r1b, r2w, r2b, y_o):
    h2 = _attn_h2(h[...], ndn[...], ndd[...], wo[...], g1[...], b1[...],
                  w1[...], bb1[...], w2[...], bb2[...], g2[...], b2[...])
    y = jax.nn.relu(_mm(h2, r0w[...]) + r0b[...])
    y = jax.nn.relu(_mm(y, r1w[...]) + r1b[...])
    y_o[...] = _mm(y, r2w[...]) + r2b[...]


def _build_node_final(interpret=False):
    return pl.pallas_call(
        _node_final_body,
        out_shape=jax.ShapeDtypeStruct((N, 1), f32),
        interpret=interpret,
    )


BE = 4000
GE = E // BE


def _rowspec(w):
    return pl.BlockSpec((BE, w), lambda i: (i, 0))


def _cspec(shape):
    return pl.BlockSpec(shape, lambda i: (0,) * len(shape))


def _edge0_body(ea, eemb, we, e_o, ep_o):
    e0 = _mm(ea[...], eemb[...])
    e_o[...] = e0
    ep_o[...] = _mm(e0, we[...])


def _build_edge0(interpret=False):
    return pl.pallas_call(
        _edge0_body,
        grid=(GE,),
        in_specs=[_rowspec(16), _cspec((16, HID)), _cspec((HID, HID))],
        out_specs=(_rowspec(HID), _rowspec(HID)),
        out_shape=(jax.ShapeDtypeStruct((E, HID), f32),) * 2,
        interpret=interpret,
    )


def _accum_stats(i, stats_o, t):
    s0 = jnp.sum(t, axis=0, keepdims=True)
    s1 = jnp.sum(t * t, axis=0, keepdims=True)
    blk = jnp.concatenate([s0, s1, jnp.zeros((6, HID), f32)], axis=0)

    @pl.when(i == 0)
    def _():
        stats_o[...] = blk

    @pl.when(i > 0)
    def _():
        stats_o[...] = stats_o[...] + blk


def _wb_rows(sc, denbt, denb):
    s8 = _mm(sc, denbt)
    w = jnp.exp(jnp.minimum(jnp.maximum(s8, -5.0), 5.0))
    return _mm(w, denb)


def _edge_p1_body(e, sc, woe, denbt, denb, t_o, stats_o, wb_o):
    i = pl.program_id(0)
    t = e[...] + _mm(sc[...], woe[...])
    t_o[...] = t
    _accum_stats(i, stats_o, t)
    wb_o[...] = _wb_rows(sc[...], denbt[...], denb[...])


def _build_edge_p1(interpret=False):
    return pl.pallas_call(
        _edge_p1_body,
        grid=(GE,),
        in_specs=[_rowspec(HID), _rowspec(HID), _cspec((HID, HID)),
                  _cspec((HID, HEADS)), _cspec((HEADS, HID))],
        out_specs=(_rowspec(HID), _cspec((8, HID)), _rowspec(HID)),
        out_shape=(jax.ShapeDtypeStruct((E, HID), f32),
                   jax.ShapeDtypeStruct((8, HID), f32),
                   jax.ShapeDtypeStruct((E, HID), f32)),
        interpret=interpret,
    )


def _mid_body(sc, denbt, denb, wb_o):
    wb_o[...] = _wb_rows(sc[...], denbt[...], denb[...])


def _build_mid(interpret=False):
    return pl.pallas_call(
        _mid_body,
        grid=(GE,),
        in_specs=[_rowspec(HID), _cspec((HID, HEADS)), _cspec((HEADS, HID))],
        out_specs=_rowspec(HID),
        out_shape=jax.ShapeDtypeStruct((E, HID), f32),
        interpret=interpret,
    )


def _apply_stats(t, stats, g, b):
    mean = stats[0:1, :] * (1.0 / E)
    var = stats[1:2, :] * (1.0 / E) - mean * mean
    return (t - mean) * lax.rsqrt(var + 1e-5) * g + b


def _edge_p2_body(t, stats, ge1, be1, we1, bbe1, we2, bbe2, u_o, stats_o):
    i = pl.program_id(0)
    e1 = _apply_stats(t[...], stats[...], ge1[...], be1[...])
    u = e1 + _mm(jax.nn.relu(_mm(e1, we1[...]) + bbe1[...]), we2[...]) + bbe2[...]
    u_o[...] = u
    _accum_stats(i, stats_o, u)


def _build_edge_p2(interpret=False):
    return pl.pallas_call(
        _edge_p2_body,
        grid=(GE,),
        in_specs=[_rowspec(HID), _cspec((8, HID)), _cspec((1, HID)),
                  _cspec((1, HID)), _cspec((HID, 2 * HID)), _cspec((1, 2 * HID)),
                  _cspec((2 * HID, HID)), _cspec((1, HID))],
        out_specs=(_rowspec(HID), _cspec((8, HID))),
        out_shape=(jax.ShapeDtypeStruct((E, HID), f32),
                   jax.ShapeDtypeStruct((8, HID), f32)),
        interpret=interpret,
    )


def _edge_p3_body(u, stats, ge2, be2, wen, e_o, ep_o):
    e2 = _apply_stats(u[...], stats[...], ge2[...], be2[...])
    e_o[...] = e2
    ep_o[...] = _mm(e2, wen[...])


def _edge_p3_noe_body(u, stats, ge2, be2, wen, ep_o):
    e2 = _apply_stats(u[...], stats[...], ge2[...], be2[...])
    ep_o[...] = _mm(e2, wen[...])


def _build_edge_p3(emit_e, interpret=False):
    specs = [_rowspec(HID), _cspec((8, HID)), _cspec((1, HID)),
             _cspec((1, HID)), _cspec((HID, HID))]
    if emit_e:
        return pl.pallas_call(
            _edge_p3_body,
            grid=(GE,),
            in_specs=specs,
            out_specs=(_rowspec(HID), _rowspec(HID)),
            out_shape=(jax.ShapeDtypeStruct((E, HID), f32),) * 2,
            interpret=interpret,
        )
    return pl.pallas_call(
        _edge_p3_noe_body,
        grid=(GE,),
        in_specs=specs,
        out_specs=(_rowspec(HID),),
        out_shape=(jax.ShapeDtypeStruct((E, HID), f32),),
        interpret=interpret,
    )


_node0 = _build_node0()
_node_mid = _build_node_mid()
_node_final = _build_node_final()
_edge0 = _build_edge0()
_edge_p1 = _build_edge_p1()
_mid = _build_mid()
_edge_p2 = _build_edge_p2()
_edge_p3 = _build_edge_p3(True)
_edge_p3_noe = _build_edge_p3(False)


def _row(v):
    return v.reshape(1, -1)


def kernel(x, edge_attr, params, edge_index):
    src = edge_index[0]
    dst = edge_index[1]
    denb = jnp.asarray(_DENB_NP)
    denbt = jnp.asarray(_DENB_NP.T)
    p0 = params["layers"][0]
    h, Q, K, V = _node0(x, params["node_emb"], p0["Wq"], p0["Wk"], p0["Wv"])
    e, Ep = _edge0(edge_attr, params["edge_emb"], p0["We"])
    for i in range(LAYERS):
        li = params["layers"][i]
        node_args = (li["Wo"], _row(li["g1"]), _row(li["b1"]),
                     li["W1"], _row(li["bb1"]), li["W2"], _row(li["bb2"]),
                     _row(li["g2"]), _row(li["b2"]))
        eout = _build_sc_a()(Q, K, Ep, src, dst)
        if i < LAYERS - 1:
            t, st, wb = _edge_p1(e, eout, li["Woe"], denbt, denb)
        else:
            wb = _mid(eout, denbt, denb)
        ndn, ndd = _build_sc_b()(V, wb, src, dst)
        if i < LAYERS - 1:
            nl = params["layers"][i + 1]
            h, Q, K, V = _node_mid(h, ndn, ndd, *node_args,
                                   nl["Wq"], nl["Wk"], nl["Wv"])
            u, st2 = _edge_p2(t, st, _row(li["ge1"]), _row(li["be1"]),
                              li["We1"], _row(li["bbe1"]),
                              li["We2"], _row(li["bbe2"]))
            if i < LAYERS - 2:
                e, Ep = _edge_p3(u, st2, _row(li["ge2"]), _row(li["be2"]),
                                 nl["We"])
            else:
                (Ep,) = _edge_p3_noe(u, st2, _row(li["ge2"]), _row(li["be2"]),
                                     nl["We"])
        else:
            y = _node_final(h, ndn, ndd, *node_args,
                            params["r0W"], _row(params["r0b"]),
                            params["r1W"], _row(params["r1b"]),
                            params["r2W"], _row(params["r2b"]))
    return y


# speedup vs baseline: 22.9975x; 1.1874x over previous
import functools

import numpy as np
import jax
import jax.numpy as jnp
from jax import lax
from jax.experimental import pallas as pl
from jax.experimental.pallas import tpu as pltpu
from jax.experimental.pallas import tpu_sc as plsc

N = 10000
E = 320000
HID = 128
HEADS = 8
DH = 16
LAYERS = 4

f32 = jnp.float32

NC = 2
NS = 16
NW = NC * NS
EW = E // NW
C = 80
G = EW // C
NP = 10240
RPT = NP // NS

_DENB_NP = np.zeros((HEADS, HID), np.float32)
for _h in range(HEADS):
    _DENB_NP[_h, _h * DH:(_h + 1) * DH] = 1.0


def _mm(a, b):
    return jnp.dot(a, b, preferred_element_type=f32)


def _bn(v, g, b):
    m = jnp.mean(v, axis=0, keepdims=True)
    var = jnp.mean((v - m) ** 2, axis=0, keepdims=True)
    return (v - m) * lax.rsqrt(var + 1e-5) * g + b


@functools.lru_cache(maxsize=None)
def _build_sc_a():
    mesh = plsc.VectorSubcoreMesh(core_axis_name="c", subcore_axis_name="s")

    def body(q_hbm, k_hbm, ep_hbm, src_hbm, dst_hbm, eout_hbm,
             srcv, dstv, qb, kb, epb, sb, semi, semg):
        cid = lax.axis_index("c")
        sid = lax.axis_index("s")
        wid = sid * NC + cid
        base_w = wid * EW

        def bs(g):
            return pl.multiple_of(base_w + g * C, 8)

        def idx_start(g, p):
            pltpu.make_async_copy(src_hbm.at[pl.ds(bs(g), C)], srcv.at[p],
                                  semi.at[p]).start()
            pltpu.make_async_copy(dst_hbm.at[pl.ds(bs(g), C)], dstv.at[p],
                                  semi.at[p]).start()

        def idx_wait(g, p):
            pltpu.make_async_copy(src_hbm.at[pl.ds(bs(g), C)], srcv.at[p],
                                  semi.at[p]).wait()
            pltpu.make_async_copy(dst_hbm.at[pl.ds(bs(g), C)], dstv.at[p],
                                  semi.at[p]).wait()

        def gat_start(g, p):
            pltpu.make_async_copy(q_hbm.at[dstv.at[p]], qb.at[p],
                                  semg.at[p]).start()
            pltpu.make_async_copy(k_hbm.at[srcv.at[p]], kb.at[p],
                                  semg.at[p]).start()
            pltpu.make_async_copy(ep_hbm.at[pl.ds(bs(g), C)], epb.at[p],
                                  semg.at[p]).start()

        def gat_wait(g, p):
            pltpu.make_async_copy(q_hbm.at[dstv.at[p]], qb.at[p],
                                  semg.at[p]).wait()
            pltpu.make_async_copy(k_hbm.at[srcv.at[p]], kb.at[p],
                                  semg.at[p]).wait()
            pltpu.make_async_copy(ep_hbm.at[pl.ds(bs(g), C)], epb.at[p],
                                  semg.at[p]).wait()

        idx_start(0, 0)
        idx_wait(0, 0)
        gat_start(0, 0)
        idx_start(1, 1)

        def chunk(g, carry):
            p = lax.rem(g, 2)
            q = 1 - p
            gat_wait(g, p)

            @pl.when(g < G - 1)
            def _():
                idx_wait(g + 1, q)
                gat_start(g + 1, q)

            def edge(i, c2):
                for h in range(HEADS):
                    sl = pl.ds(h * DH, DH)
                    sb[i, sl] = qb[p, i, sl] * kb[p, i, sl] * 0.25 * epb[p, i, sl]
                return c2

            lax.fori_loop(0, C, edge, 0)
            pltpu.sync_copy(sb, eout_hbm.at[pl.ds(bs(g), C)])

            @pl.when(g < G - 2)
            def _():
                idx_start(g + 2, p)

            return carry

        lax.fori_loop(0, G, chunk, 0)

    return pl.kernel(
        body,
        out_type=jax.ShapeDtypeStruct((E, HID), f32),
        mesh=mesh,
        scratch_types=[
            pltpu.VMEM((2, C), jnp.int32),
            pltpu.VMEM((2, C), jnp.int32),
            pltpu.VMEM((2, C, HID), f32),
            pltpu.VMEM((2, C, HID), f32),
            pltpu.VMEM((2, C, HID), f32),
            pltpu.VMEM((C, HID), f32),
            pltpu.SemaphoreType.DMA((2,)),
            pltpu.SemaphoreType.DMA((2,)),
        ],
    )


@functools.lru_cache(maxsize=None)
def _build_sc_b():
    mesh = plsc.VectorSubcoreMesh(core_axis_name="c", subcore_axis_name="s")

    def body(v_hbm, wb_hbm, src_hbm, dst_hbm, ndn_hbm, ndd_hbm,
             srcv, dstv, vb, wbb, acc, semi, semg):
        cid = lax.axis_index("c")
        sid = lax.axis_index("s")
        wid = sid * NC + cid
        base_w = wid * EW
        z16 = jnp.zeros((16,), f32)

        def bs(g):
            return pl.multiple_of(base_w + g * C, 8)

        def zero_ndb():
            def zrow(r, carry):
                for j in range(HID // 16):
                    wbb[0, r, pl.ds(j * 16, 16)] = z16
                return carry

            lax.fori_loop(0, C, zrow, 0)

        def zero_acc():
            for j in range(RPT // C):
                pltpu.sync_copy(wbb.at[0],
                                acc.at[pl.ds(sid * RPT + j * C, C)])

        zero_ndb()
        zero_acc()
        plsc.subcore_barrier()

        def idx_start(g, p, with_src):
            if with_src:
                pltpu.make_async_copy(src_hbm.at[pl.ds(bs(g), C)],
                                      srcv.at[p], semi.at[p]).start()
            pltpu.make_async_copy(dst_hbm.at[pl.ds(bs(g), C)], dstv.at[p],
                                  semi.at[p]).start()

        def idx_wait(g, p, with_src):
            if with_src:
                pltpu.make_async_copy(src_hbm.at[pl.ds(bs(g), C)],
                                      srcv.at[p], semi.at[p]).wait()
            pltpu.make_async_copy(dst_hbm.at[pl.ds(bs(g), C)], dstv.at[p],
                                  semi.at[p]).wait()

        def gat_start(g, p, with_v):
            if with_v:
                pltpu.make_async_copy(v_hbm.at[srcv.at[p]], vb.at[p],
                                      semg.at[p]).start()
            pltpu.make_async_copy(wb_hbm.at[pl.ds(bs(g), C)], wbb.at[p],
                                  semg.at[p]).start()

        def gat_wait(g, p, with_v):
            if with_v:
                pltpu.make_async_copy(v_hbm.at[srcv.at[p]], vb.at[p],
                                      semg.at[p]).wait()
            pltpu.make_async_copy(wb_hbm.at[pl.ds(bs(g), C)], wbb.at[p],
                                  semg.at[p]).wait()

        def phase(with_v, out_hbm):
            idx_start(0, 0, with_v)
            idx_wait(0, 0, with_v)
            gat_start(0, 0, with_v)
            idx_start(1, 1, with_v)

            def chunk(g, carry):
                p = lax.rem(g, 2)
                q = 1 - p
                gat_wait(g, p, with_v)

                @pl.when(g < G - 1)
                def _():
                    idx_wait(g + 1, q, with_v)
                    gat_start(g + 1, q, with_v)

                if with_v:
                    def edge(i, c2):
                        for h in range(HEADS):
                            sl = pl.ds(h * DH, DH)
                            vb[p, i, sl] = vb[p, i, sl] * wbb[p, i, sl]
                        return c2

                    lax.fori_loop(0, C, edge, 0)
                    pltpu.sync_copy(vb.at[p], acc.at[dstv.at[p]], add=True)
                else:
                    pltpu.sync_copy(wbb.at[p], acc.at[dstv.at[p]], add=True)

                @pl.when(g < G - 2)
                def _():
                    idx_start(g + 2, p, with_v)

                return carry

            lax.fori_loop(0, G, chunk, 0)
            plsc.subcore_barrier()
            pltpu.sync_copy(acc.at[pl.ds(sid * RPT, RPT)],
                            out_hbm.at[cid, pl.ds(sid * RPT, RPT)])

        phase(True, ndn_hbm)
        zero_ndb()
        zero_acc()
        plsc.subcore_barrier()
        phase(False, ndd_hbm)

    return pl.kernel(
        body,
        out_type=(jax.ShapeDtypeStruct((NC, NP, HID), f32),
                  jax.ShapeDtypeStruct((NC, NP, HID), f32)),
        mesh=mesh,
        scratch_types=[
            pltpu.VMEM((2, C), jnp.int32),
            pltpu.VMEM((2, C), jnp.int32),
            pltpu.VMEM((2, C, HID), f32),
            pltpu.VMEM((2, C, HID), f32),
            pltpu.VMEM_SHARED((NP, HID), f32),
            pltpu.SemaphoreType.DMA((2,)),
            pltpu.SemaphoreType.DMA((2,)),
        ],
    )


def _node0_body(x, ne, wq, wk, wv, h_o, q_o, k_o, v_o):
    h = _mm(x[...], ne[...])
    h_o[...] = h
    q_o[...] = _mm(h, wq[...])
    k_o[...] = _mm(h, wk[...])
    v_o[...] = _mm(h, wv[...])


def _build_node0(interpret=False):
    return pl.pallas_call(
        _node0_body,
        out_shape=(jax.ShapeDtypeStruct((N, HID), f32),) * 4,
        interpret=interpret,
    )


def _attn_h2(h, ndn, ndd, wo, g1, b1, w1, bb1, w2, bb2, g2, b2):
    num = ndn[0, :N] + ndn[1, :N]
    den = ndd[0, :N] + ndd[1, :N]
    attn = num / (den + 1e-6)
    t = h + _mm(attn, wo)
    h1 = _bn(t, g1, b1)
    ff = _mm(jax.nn.relu(_mm(h1, w1) + bb1), w2) + bb2
    return _bn(h1 + ff, g2, b2)


def _node_mid_body(h, ndn, ndd, wo, g1, b1, w1, bb1, w2, bb2, g2, b2,
                   wq, wk, wv, h_o, q_o, k_o, v_o):
    h2 = _attn_h2(h[...], ndn[...], ndd[...], wo[...], g1[...], b1[...],
                  w1[...], bb1[...], w2[...], bb2[...], g2[...], b2[...])
    h_o[...] = h2
    q_o[...] = _mm(h2, wq[...])
    k_o[...] = _mm(h2, wk[...])
    v_o[...] = _mm(h2, wv[...])


def _build_node_mid(interpret=False):
    return pl.pallas_call(
        _node_mid_body,
        out_shape=(jax.ShapeDtypeStruct((N, HID), f32),) * 4,
        interpret=interpret,
    )


def _node_final_body(h, ndn, ndd, wo, g1, b1, w1, bb1, w2, bb2, g2, b2,
                     r0w, r0b, r1w, r1b, r2w, r2b, y_o):
    h2 = _attn_h2(h[...], ndn[...], ndd[...], wo[...], g1[...], b1[...],
                  w1[...], bb1[...], w2[...], bb2[...], g2[...], b2[...])
    y = jax.nn.relu(_mm(h2, r0w[...]) + r0b[...])
    y = jax.nn.relu(_mm(y, r1w[...]) + r1b[...])
    y_o[...] = _mm(y, r2w[...]) + r2b[...]


def _build_node_final(interpret=False):
    return pl.pallas_call(
        _node_final_body,
        out_shape=jax.ShapeDtypeStruct((N, 1), f32),
        interpret=interpret,
    )


BE = 4000
GE = E // BE


def _rowspec(w):
    return pl.BlockSpec((BE, w), lambda i: (i, 0))


def _cspec(shape):
    return pl.BlockSpec(shape, lambda i: (0,) * len(shape))


def _edge0_body(ea, eemb, we, e_o, ep_o):
    e0 = _mm(ea[...], eemb[...])
    e_o[...] = e0
    ep_o[...] = _mm(e0, we[...])


def _build_edge0(interpret=False):
    return pl.pallas_call(
        _edge0_body,
        grid=(GE,),
        in_specs=[_rowspec(16), _cspec((16, HID)), _cspec((HID, HID))],
        out_specs=(_rowspec(HID), _rowspec(HID)),
        out_shape=(jax.ShapeDtypeStruct((E, HID), f32),) * 2,
        interpret=interpret,
    )


def _accum_stats(i, stats_o, t):
    s0 = jnp.sum(t, axis=0, keepdims=True)
    s1 = jnp.sum(t * t, axis=0, keepdims=True)
    blk = jnp.concatenate([s0, s1, jnp.zeros((6, HID), f32)], axis=0)

    @pl.when(i == 0)
    def _():
        stats_o[...] = blk

    @pl.when(i > 0)
    def _():
        stats_o[...] = stats_o[...] + blk


def _wb_rows(sc, denbt, denb):
    s8 = _mm(sc, denbt)
    w = jnp.exp(jnp.minimum(jnp.maximum(s8, -5.0), 5.0))
    return _mm(w, denb)


def _edge_p1_body(e, sc, woe, denbt, denb, t_o, stats_o, wb_o):
    i = pl.program_id(0)
    t = e[...] + _mm(sc[...], woe[...])
    t_o[...] = t
    _accum_stats(i, stats_o, t)
    wb_o[...] = _wb_rows(sc[...], denbt[...], denb[...])


def _build_edge_p1(interpret=False):
    return pl.pallas_call(
        _edge_p1_body,
        grid=(GE,),
        in_specs=[_rowspec(HID), _rowspec(HID), _cspec((HID, HID)),
                  _cspec((HID, HEADS)), _cspec((HEADS, HID))],
        out_specs=(_rowspec(HID), _cspec((8, HID)), _rowspec(HID)),
        out_shape=(jax.ShapeDtypeStruct((E, HID), f32),
                   jax.ShapeDtypeStruct((8, HID), f32),
                   jax.ShapeDtypeStruct((E, HID), f32)),
        interpret=interpret,
    )


def _mid_body(sc, denbt, denb, wb_o):
    wb_o[...] = _wb_rows(sc[...], denbt[...], denb[...])


def _build_mid(interpret=False):
    return pl.pallas_call(
        _mid_body,
        grid=(GE,),
        in_specs=[_rowspec(HID), _cspec((HID, HEADS)), _cspec((HEADS, HID))],
        out_specs=_rowspec(HID),
        out_shape=jax.ShapeDtypeStruct((E, HID), f32),
        interpret=interpret,
    )


def _apply_stats(t, stats, g, b):
    mean = stats[0:1, :] * (1.0 / E)
    var = stats[1:2, :] * (1.0 / E) - mean * mean
    return (t - mean) * lax.rsqrt(var + 1e-5) * g + b


def _edge_p2_body(t, stats, ge1, be1, we1, bbe1, we2, bbe2, u_o, stats_o):
    i = pl.program_id(0)
    e1 = _apply_stats(t[...], stats[...], ge1[...], be1[...])
    u = e1 + _mm(jax.nn.relu(_mm(e1, we1[...]) + bbe1[...]), we2[...]) + bbe2[...]
    u_o[...] = u
    _accum_stats(i, stats_o, u)


def _build_edge_p2(interpret=False):
    return pl.pallas_call(
        _edge_p2_body,
        grid=(GE,),
        in_specs=[_rowspec(HID), _cspec((8, HID)), _cspec((1, HID)),
                  _cspec((1, HID)), _cspec((HID, 2 * HID)), _cspec((1, 2 * HID)),
                  _cspec((2 * HID, HID)), _cspec((1, HID))],
        out_specs=(_rowspec(HID), _cspec((8, HID))),
        out_shape=(jax.ShapeDtypeStruct((E, HID), f32),
                   jax.ShapeDtypeStruct((8, HID), f32)),
        interpret=interpret,
    )


def _edge_p3_body(u, stats, ge2, be2, wen, e_o, ep_o):
    e2 = _apply_stats(u[...], stats[...], ge2[...], be2[...])
    e_o[...] = e2
    ep_o[...] = _mm(e2, wen[...])


def _edge_p3_noe_body(u, stats, ge2, be2, wen, ep_o):
    e2 = _apply_stats(u[...], stats[...], ge2[...], be2[...])
    ep_o[...] = _mm(e2, wen[...])


def _build_edge_p3(emit_e, interpret=False):
    specs = [_rowspec(HID), _cspec((8, HID)), _cspec((1, HID)),
             _cspec((1, HID)), _cspec((HID, HID))]
    if emit_e:
        return pl.pallas_call(
            _edge_p3_body,
            grid=(GE,),
            in_specs=specs,
            out_specs=(_rowspec(HID), _rowspec(HID)),
            out_shape=(jax.ShapeDtypeStruct((E, HID), f32),) * 2,
            interpret=interpret,
        )
    return pl.pallas_call(
        _edge_p3_noe_body,
        grid=(GE,),
        in_specs=specs,
        out_specs=(_rowspec(HID),),
        out_shape=(jax.ShapeDtypeStruct((E, HID), f32),),
        interpret=interpret,
    )


_node0 = _build_node0()
_node_mid = _build_node_mid()
_node_final = _build_node_final()
_edge0 = _build_edge0()
_edge_p1 = _build_edge_p1()
_mid = _build_mid()
_edge_p2 = _build_edge_p2()
_edge_p3 = _build_edge_p3(True)
_edge_p3_noe = _build_edge_p3(False)


def _row(v):
    return v.reshape(1, -1)


def kernel(x, edge_attr, params, edge_index):
    src = edge_index[0]
    dst = edge_index[1]
    denb = jnp.asarray(_DENB_NP)
    denbt = jnp.asarray(_DENB_NP.T)
    p0 = params["layers"][0]
    h, Q, K, V = _node0(x, params["node_emb"], p0["Wq"], p0["Wk"], p0["Wv"])
    e, Ep = _edge0(edge_attr, params["edge_emb"], p0["We"])
    for i in range(LAYERS):
        li = params["layers"][i]
        node_args = (li["Wo"], _row(li["g1"]), _row(li["b1"]),
                     li["W1"], _row(li["bb1"]), li["W2"], _row(li["bb2"]),
                     _row(li["g2"]), _row(li["b2"]))
        eout = _build_sc_a()(Q, K, Ep, src, dst)
        if i < LAYERS - 1:
            t, st, wb = _edge_p1(e, eout, li["Woe"], denbt, denb)
        else:
            wb = _mid(eout, denbt, denb)
        ndn, ndd = _build_sc_b()(V, wb, src, dst)
        if i < LAYERS - 1:
            nl = params["layers"][i + 1]
            h, Q, K, V = _node_mid(h, ndn, ndd, *node_args,
                                   nl["Wq"], nl["Wk"], nl["Wv"])
            u, st2 = _edge_p2(t, st, _row(li["ge1"]), _row(li["be1"]),
                              li["We1"], _row(li["bbe1"]),
                              li["We2"], _row(li["bbe2"]))
            if i < LAYERS - 2:
                e, Ep = _edge_p3(u, st2, _row(li["ge2"]), _row(li["be2"]),
                                 nl["We"])
            else:
                (Ep,) = _edge_p3_noe(u, st2, _row(li["ge2"]), _row(li["be2"]),
                                     nl["We"])
        else:
            y = _node_final(h, ndn, ndd, *node_args,
                            params["r0W"], _row(params["r0b"]),
                            params["r1W"], _row(params["r1b"]),
                            params["r2W"], _row(params["r2b"]))
    return y


# speedup vs baseline: 24.3679x; 1.0596x over previous
import functools

import numpy as np
import jax
import jax.numpy as jnp
from jax import lax
from jax.experimental import pallas as pl
from jax.experimental.pallas import tpu as pltpu
from jax.experimental.pallas import tpu_sc as plsc

N = 10000
E = 320000
HID = 128
HEADS = 8
DH = 16
LAYERS = 4

f32 = jnp.float32

NC = 2
NS = 16
NW = NC * NS
EW = E // NW
C = 80
G = EW // C
NP = 10240
RPT = NP // NS

_DENB_NP = np.zeros((HEADS, HID), np.float32)
for _h in range(HEADS):
    _DENB_NP[_h, _h * DH:(_h + 1) * DH] = 1.0


def _mm(a, b):
    return jnp.dot(a, b, preferred_element_type=f32)


def _bn(v, g, b):
    m = jnp.mean(v, axis=0, keepdims=True)
    var = jnp.mean((v - m) ** 2, axis=0, keepdims=True)
    return (v - m) * lax.rsqrt(var + 1e-5) * g + b


@functools.lru_cache(maxsize=None)
def _build_sc_a():
    mesh = plsc.VectorSubcoreMesh(core_axis_name="c", subcore_axis_name="s")

    def body(q_hbm, k_hbm, ep_hbm, src_hbm, dst_hbm, eout_hbm,
             srcv, dstv, qb, kb, epb, sb, semi, semg, semo):
        cid = lax.axis_index("c")
        sid = lax.axis_index("s")
        wid = sid * NC + cid
        base_w = wid * EW

        def bs(g):
            return pl.multiple_of(base_w + g * C, 8)

        def idx_start(g, p):
            pltpu.make_async_copy(src_hbm.at[pl.ds(bs(g), C)], srcv.at[p],
                                  semi.at[p]).start()
            pltpu.make_async_copy(dst_hbm.at[pl.ds(bs(g), C)], dstv.at[p],
                                  semi.at[p]).start()

        def idx_wait(g, p):
            pltpu.make_async_copy(src_hbm.at[pl.ds(bs(g), C)], srcv.at[p],
                                  semi.at[p]).wait()
            pltpu.make_async_copy(dst_hbm.at[pl.ds(bs(g), C)], dstv.at[p],
                                  semi.at[p]).wait()

        def gat_start(g, p):
            pltpu.make_async_copy(q_hbm.at[dstv.at[p]], qb.at[p],
                                  semg.at[p]).start()
            pltpu.make_async_copy(k_hbm.at[srcv.at[p]], kb.at[p],
                                  semg.at[p]).start()
            pltpu.make_async_copy(ep_hbm.at[pl.ds(bs(g), C)], epb.at[p],
                                  semg.at[p]).start()

        def gat_wait(g, p):
            pltpu.make_async_copy(q_hbm.at[dstv.at[p]], qb.at[p],
                                  semg.at[p]).wait()
            pltpu.make_async_copy(k_hbm.at[srcv.at[p]], kb.at[p],
                                  semg.at[p]).wait()
            pltpu.make_async_copy(ep_hbm.at[pl.ds(bs(g), C)], epb.at[p],
                                  semg.at[p]).wait()

        idx_start(0, 0)
        idx_wait(0, 0)
        gat_start(0, 0)
        idx_start(1, 1)

        def chunk(g, carry):
            p = lax.rem(g, 2)
            q = 1 - p
            gat_wait(g, p)

            @pl.when(g < G - 1)
            def _():
                idx_wait(g + 1, q)
                gat_start(g + 1, q)

            @pl.when(g >= 1)
            def _():
                pltpu.make_async_copy(sb.at[q], eout_hbm.at[pl.ds(bs(g - 1), C)],
                                      semo.at[q]).wait()

            def edge(i, c2):
                for h in range(HEADS):
                    sl = pl.ds(h * DH, DH)
                    sb[p, i, sl] = qb[p, i, sl] * kb[p, i, sl] * 0.25 * epb[p, i, sl]
                return c2

            lax.fori_loop(0, C, edge, 0)
            pltpu.make_async_copy(sb.at[p], eout_hbm.at[pl.ds(bs(g), C)],
                                  semo.at[p]).start()

            @pl.when(g < G - 2)
            def _():
                idx_start(g + 2, p)

            return carry

        lax.fori_loop(0, G, chunk, 0)
        pltpu.make_async_copy(sb.at[lax.rem(G - 1, 2)],
                              eout_hbm.at[pl.ds(bs(G - 1), C)],
                              semo.at[lax.rem(G - 1, 2)]).wait()

    return pl.kernel(
        body,
        out_type=jax.ShapeDtypeStruct((E, HID), f32),
        mesh=mesh,
        scratch_types=[
            pltpu.VMEM((2, C), jnp.int32),
            pltpu.VMEM((2, C), jnp.int32),
            pltpu.VMEM((2, C, HID), f32),
            pltpu.VMEM((2, C, HID), f32),
            pltpu.VMEM((2, C, HID), f32),
            pltpu.VMEM((2, C, HID), f32),
            pltpu.SemaphoreType.DMA((2,)),
            pltpu.SemaphoreType.DMA((2,)),
            pltpu.SemaphoreType.DMA((2,)),
        ],
    )


@functools.lru_cache(maxsize=None)
def _build_sc_b():
    mesh = plsc.VectorSubcoreMesh(core_axis_name="c", subcore_axis_name="s")

    def body(v_hbm, wb_hbm, src_hbm, dst_hbm, ndn_hbm, ndd_hbm,
             srcv, dstv, vb, wbb, acc, semi, semg, semsc):
        cid = lax.axis_index("c")
        sid = lax.axis_index("s")
        wid = sid * NC + cid
        base_w = wid * EW
        z16 = jnp.zeros((16,), f32)

        def bs(g):
            return pl.multiple_of(base_w + g * C, 8)

        def zero_ndb():
            def zrow(r, carry):
                for j in range(HID // 16):
                    wbb[0, r, pl.ds(j * 16, 16)] = z16
                return carry

            lax.fori_loop(0, C, zrow, 0)

        def zero_acc():
            for j in range(RPT // C):
                pltpu.sync_copy(wbb.at[0],
                                acc.at[pl.ds(sid * RPT + j * C, C)])

        zero_ndb()
        zero_acc()
        plsc.subcore_barrier()

        def idx_start(g, p, with_src):
            if with_src:
                pltpu.make_async_copy(src_hbm.at[pl.ds(bs(g), C)],
                                      srcv.at[p], semi.at[p]).start()
            pltpu.make_async_copy(dst_hbm.at[pl.ds(bs(g), C)], dstv.at[p],
                                  semi.at[p]).start()

        def idx_wait(g, p, with_src):
            if with_src:
                pltpu.make_async_copy(src_hbm.at[pl.ds(bs(g), C)],
                                      srcv.at[p], semi.at[p]).wait()
            pltpu.make_async_copy(dst_hbm.at[pl.ds(bs(g), C)], dstv.at[p],
                                  semi.at[p]).wait()

        def gat_start(g, p, with_v):
            if with_v:
                pltpu.make_async_copy(v_hbm.at[srcv.at[p]], vb.at[p],
                                      semg.at[p]).start()
            pltpu.make_async_copy(wb_hbm.at[pl.ds(bs(g), C)], wbb.at[p],
                                  semg.at[p]).start()

        def gat_wait(g, p, with_v):
            if with_v:
                pltpu.make_async_copy(v_hbm.at[srcv.at[p]], vb.at[p],
                                      semg.at[p]).wait()
            pltpu.make_async_copy(wb_hbm.at[pl.ds(bs(g), C)], wbb.at[p],
                                  semg.at[p]).wait()

        def phase(with_v, out_hbm):
            idx_start(0, 0, with_v)
            idx_wait(0, 0, with_v)
            gat_start(0, 0, with_v)
            idx_start(1, 1, with_v)

            sbuf = vb if with_v else wbb

            def scat_wait(g, p):
                pltpu.make_async_copy(sbuf.at[p], acc.at[dstv.at[p]],
                                      semsc.at[p]).wait()

            def chunk(g, carry):
                p = lax.rem(g, 2)
                q = 1 - p
                gat_wait(g, p, with_v)

                @pl.when(g >= 1)
                def _():
                    scat_wait(g - 1, q)

                @pl.when(g < G - 1)
                def _():
                    idx_wait(g + 1, q, with_v)
                    gat_start(g + 1, q, with_v)

                if with_v:
                    def edge(i, c2):
                        for h in range(HEADS):
                            sl = pl.ds(h * DH, DH)
                            vb[p, i, sl] = vb[p, i, sl] * wbb[p, i, sl]
                        return c2

                    lax.fori_loop(0, C, edge, 0)
                pltpu.async_copy(sbuf.at[p], acc.at[dstv.at[p]],
                                 semsc.at[p], add=True)

                @pl.when(g < G - 2)
                def _():
                    idx_start(g + 2, p, with_v)

                return carry

            lax.fori_loop(0, G, chunk, 0)
            scat_wait(G - 1, lax.rem(G - 1, 2))
            plsc.subcore_barrier()
            pltpu.sync_copy(acc.at[pl.ds(sid * RPT, RPT)],
                            out_hbm.at[cid, pl.ds(sid * RPT, RPT)])

        phase(True, ndn_hbm)
        zero_ndb()
        zero_acc()
        plsc.subcore_barrier()
        phase(False, ndd_hbm)

    return pl.kernel(
        body,
        out_type=(jax.ShapeDtypeStruct((NC, NP, HID), f32),
                  jax.ShapeDtypeStruct((NC, NP, HID), f32)),
        mesh=mesh,
        scratch_types=[
            pltpu.VMEM((2, C), jnp.int32),
            pltpu.VMEM((2, C), jnp.int32),
            pltpu.VMEM((2, C, HID), f32),
            pltpu.VMEM((2, C, HID), f32),
            pltpu.VMEM_SHARED((NP, HID), f32),
            pltpu.SemaphoreType.DMA((2,)),
            pltpu.SemaphoreType.DMA((2,)),
            pltpu.SemaphoreType.DMA((2,)),
        ],
    )


def _node0_body(x, ne, wq, wk, wv, h_o, q_o, k_o, v_o):
    h = _mm(x[...], ne[...])
    h_o[...] = h
    q_o[...] = _mm(h, wq[...])
    k_o[...] = _mm(h, wk[...])
    v_o[...] = _mm(h, wv[...])


def _build_node0(interpret=False):
    return pl.pallas_call(
        _node0_body,
        out_shape=(jax.ShapeDtypeStruct((N, HID), f32),) * 4,
        interpret=interpret,
    )


def _attn_h2(h, ndn, ndd, wo, g1, b1, w1, bb1, w2, bb2, g2, b2):
    num = ndn[0, :N] + ndn[1, :N]
    den = ndd[0, :N] + ndd[1, :N]
    attn = num / (den + 1e-6)
    t = h + _mm(attn, wo)
    h1 = _bn(t, g1, b1)
    ff = _mm(jax.nn.relu(_mm(h1, w1) + bb1), w2) + bb2
    return _bn(h1 + ff, g2, b2)


def _node_mid_body(h, ndn, ndd, wo, g1, b1, w1, bb1, w2, bb2, g2, b2,
                   wq, wk, wv, h_o, q_o, k_o, v_o):
    h2 = _attn_h2(h[...], ndn[...], ndd[...], wo[...], g1[...], b1[...],
                  w1[...], bb1[...], w2[...], bb2[...], g2[...], b2[...])
    h_o[...] = h2
    q_o[...] = _mm(h2, wq[...])
    k_o[...] = _mm(h2, wk[...])
    v_o[...] = _mm(h2, wv[...])


def _build_node_mid(interpret=False):
    return pl.pallas_call(
        _node_mid_body,
        out_shape=(jax.ShapeDtypeStruct((N, HID), f32),) * 4,
        interpret=interpret,
    )


def _node_final_body(h, ndn, ndd, wo, g1, b1, w1, bb1, w2, bb2, g2, b2,
                     r0w, r0b, r1w, r1b, r2w, r2b, y_o):
    h2 = _attn_h2(h[...], ndn[...], ndd[...], wo[...], g1[...], b1[...],
                  w1[...], bb1[...], w2[...], bb2[...], g2[...], b2[...])
    y = jax.nn.relu(_mm(h2, r0w[...]) + r0b[...])
    y = jax.nn.relu(_mm(y, r1w[...]) + r1b[...])
    y_o[...] = _mm(y, r2w[...]) + r2b[...]


def _build_node_final(interpret=False):
    return pl.pallas_call(
        _node_final_body,
        out_shape=jax.ShapeDtypeStruct((N, 1), f32),
        interpret=interpret,
    )


BE = 4000
GE = E // BE


def _rowspec(w):
    return pl.BlockSpec((BE, w), lambda i: (i, 0))


def _cspec(shape):
    return pl.BlockSpec(shape, lambda i: (0,) * len(shape))


def _edge0_body(ea, eemb, we, e_o, ep_o):
    e0 = _mm(ea[...], eemb[...])
    e_o[...] = e0
    ep_o[...] = _mm(e0, we[...])


def _build_edge0(interpret=False):
    return pl.pallas_call(
        _edge0_body,
        grid=(GE,),
        in_specs=[_rowspec(16), _cspec((16, HID)), _cspec((HID, HID))],
        out_specs=(_rowspec(HID), _rowspec(HID)),
        out_shape=(jax.ShapeDtypeStruct((E, HID), f32),) * 2,
        interpret=interpret,
    )


def _accum_stats(i, stats_o, t):
    s0 = jnp.sum(t, axis=0, keepdims=True)
    s1 = jnp.sum(t * t, axis=0, keepdims=True)
    blk = jnp.concatenate([s0, s1, jnp.zeros((6, HID), f32)], axis=0)

    @pl.when(i == 0)
    def _():
        stats_o[...] = blk

    @pl.when(i > 0)
    def _():
        stats_o[...] = stats_o[...] + blk


def _wb_rows(sc, denbt, denb):
    s8 = _mm(sc, denbt)
    w = jnp.exp(jnp.minimum(jnp.maximum(s8, -5.0), 5.0))
    return _mm(w, denb)


def _edge_p1_body(e, sc, woe, denbt, denb, t_o, stats_o, wb_o):
    i = pl.program_id(0)
    t = e[...] + _mm(sc[...], woe[...])
    t_o[...] = t
    _accum_stats(i, stats_o, t)
    wb_o[...] = _wb_rows(sc[...], denbt[...], denb[...])


def _build_edge_p1(interpret=False):
    return pl.pallas_call(
        _edge_p1_body,
        grid=(GE,),
        in_specs=[_rowspec(HID), _rowspec(HID), _cspec((HID, HID)),
                  _cspec((HID, HEADS)), _cspec((HEADS, HID))],
        out_specs=(_rowspec(HID), _cspec((8, HID)), _rowspec(HID)),
        out_shape=(jax.ShapeDtypeStruct((E, HID), f32),
                   jax.ShapeDtypeStruct((8, HID), f32),
                   jax.ShapeDtypeStruct((E, HID), f32)),
        interpret=interpret,
    )


def _mid_body(sc, denbt, denb, wb_o):
    wb_o[...] = _wb_rows(sc[...], denbt[...], denb[...])


def _build_mid(interpret=False):
    return pl.pallas_call(
        _mid_body,
        grid=(GE,),
        in_specs=[_rowspec(HID), _cspec((HID, HEADS)), _cspec((HEADS, HID))],
        out_specs=_rowspec(HID),
        out_shape=jax.ShapeDtypeStruct((E, HID), f32),
        interpret=interpret,
    )


def _apply_stats(t, stats, g, b):
    mean = stats[0:1, :] * (1.0 / E)
    var = stats[1:2, :] * (1.0 / E) - mean * mean
    return (t - mean) * lax.rsqrt(var + 1e-5) * g + b


def _edge_p2_body(t, stats, ge1, be1, we1, bbe1, we2, bbe2, u_o, stats_o):
    i = pl.program_id(0)
    e1 = _apply_stats(t[...], stats[...], ge1[...], be1[...])
    u = e1 + _mm(jax.nn.relu(_mm(e1, we1[...]) + bbe1[...]), we2[...]) + bbe2[...]
    u_o[...] = u
    _accum_stats(i, stats_o, u)


def _build_edge_p2(interpret=False):
    return pl.pallas_call(
        _edge_p2_body,
        grid=(GE,),
        in_specs=[_rowspec(HID), _cspec((8, HID)), _cspec((1, HID)),
                  _cspec((1, HID)), _cspec((HID, 2 * HID)), _cspec((1, 2 * HID)),
                  _cspec((2 * HID, HID)), _cspec((1, HID))],
        out_specs=(_rowspec(HID), _cspec((8, HID))),
        out_shape=(jax.ShapeDtypeStruct((E, HID), f32),
                   jax.ShapeDtypeStruct((8, HID), f32)),
        interpret=interpret,
    )


def _edge_p3_body(u, stats, ge2, be2, wen, e_o, ep_o):
    e2 = _apply_stats(u[...], stats[...], ge2[...], be2[...])
    e_o[...] = e2
    ep_o[...] = _mm(e2, wen[...])


def _edge_p3_noe_body(u, stats, ge2, be2, wen, ep_o):
    e2 = _apply_stats(u[...], stats[...], ge2[...], be2[...])
    ep_o[...] = _mm(e2, wen[...])


def _build_edge_p3(emit_e, interpret=False):
    specs = [_rowspec(HID), _cspec((8, HID)), _cspec((1, HID)),
             _cspec((1, HID)), _cspec((HID, HID))]
    if emit_e:
        return pl.pallas_call(
            _edge_p3_body,
            grid=(GE,),
            in_specs=specs,
            out_specs=(_rowspec(HID), _rowspec(HID)),
            out_shape=(jax.ShapeDtypeStruct((E, HID), f32),) * 2,
            interpret=interpret,
        )
    return pl.pallas_call(
        _edge_p3_noe_body,
        grid=(GE,),
        in_specs=specs,
        out_specs=(_rowspec(HID),),
        out_shape=(jax.ShapeDtypeStruct((E, HID), f32),),
        interpret=interpret,
    )


_node0 = _build_node0()
_node_mid = _build_node_mid()
_node_final = _build_node_final()
_edge0 = _build_edge0()
_edge_p1 = _build_edge_p1()
_mid = _build_mid()
_edge_p2 = _build_edge_p2()
_edge_p3 = _build_edge_p3(True)
_edge_p3_noe = _build_edge_p3(False)


def _row(v):
    return v.reshape(1, -1)


def kernel(x, edge_attr, params, edge_index):
    src = edge_index[0]
    dst = edge_index[1]
    denb = jnp.asarray(_DENB_NP)
    denbt = jnp.asarray(_DENB_NP.T)
    p0 = params["layers"][0]
    h, Q, K, V = _node0(x, params["node_emb"], p0["Wq"], p0["Wk"], p0["Wv"])
    e, Ep = _edge0(edge_attr, params["edge_emb"], p0["We"])
    for i in range(LAYERS):
        li = params["layers"][i]
        node_args = (li["Wo"], _row(li["g1"]), _row(li["b1"]),
                     li["W1"], _row(li["bb1"]), li["W2"], _row(li["bb2"]),
                     _row(li["g2"]), _row(li["b2"]))
        eout = _build_sc_a()(Q, K, Ep, src, dst)
        if i < LAYERS - 1:
            t, st, wb = _edge_p1(e, eout, li["Woe"], denbt, denb)
        else:
            wb = _mid(eout, denbt, denb)
        ndn, ndd = _build_sc_b()(V, wb, src, dst)
        if i < LAYERS - 1:
            nl = params["layers"][i + 1]
            h, Q, K, V = _node_mid(h, ndn, ndd, *node_args,
                                   nl["Wq"], nl["Wk"], nl["Wv"])
            u, st2 = _edge_p2(t, st, _row(li["ge1"]), _row(li["be1"]),
                              li["We1"], _row(li["bbe1"]),
                              li["We2"], _row(li["bbe2"]))
            if i < LAYERS - 2:
                e, Ep = _edge_p3(u, st2, _row(li["ge2"]), _row(li["be2"]),
                                 nl["We"])
            else:
                (Ep,) = _edge_p3_noe(u, st2, _row(li["ge2"]), _row(li["be2"]),
                                     nl["We"])
        else:
            y = _node_final(h, ndn, ndd, *node_args,
                            params["r0W"], _row(params["r0b"]),
                            params["r1W"], _row(params["r1b"]),
                            params["r2W"], _row(params["r2b"]))
    return y


# speedup vs baseline: 27.2067x; 1.1165x over previous
import functools

import numpy as np
import jax
import jax.numpy as jnp
from jax import lax
from jax.experimental import pallas as pl
from jax.experimental.pallas import tpu as pltpu
from jax.experimental.pallas import tpu_sc as plsc

N = 10000
E = 320000
HID = 128
HEADS = 8
DH = 16
LAYERS = 4

f32 = jnp.float32

NC = 2
NS = 16
NW = NC * NS
EW = E // NW
C = 80
G = EW // C
NP = 10240
RPT = NP // NS

_DENB_NP = np.zeros((HEADS, HID), np.float32)
for _h in range(HEADS):
    _DENB_NP[_h, _h * DH:(_h + 1) * DH] = 1.0


def _mm(a, b):
    return jnp.dot(a, b, preferred_element_type=f32)


def _bn(v, g, b):
    m = jnp.mean(v, axis=0, keepdims=True)
    var = jnp.mean((v - m) ** 2, axis=0, keepdims=True)
    return (v - m) * lax.rsqrt(var + 1e-5) * g + b


@functools.lru_cache(maxsize=None)
def _build_sc_a():
    mesh = plsc.VectorSubcoreMesh(core_axis_name="c", subcore_axis_name="s")

    def body(q_hbm, k_hbm, src_hbm, dst_hbm, qk_hbm,
             srcv, dstv, qb, kb, semi, semg, semo):
        cid = lax.axis_index("c")
        sid = lax.axis_index("s")
        wid = sid * NC + cid
        base_w = wid * EW

        def bs(g):
            return pl.multiple_of(base_w + g * C, 8)

        def idx_start(g, p):
            pltpu.make_async_copy(src_hbm.at[pl.ds(bs(g), C)], srcv.at[p],
                                  semi.at[p]).start()
            pltpu.make_async_copy(dst_hbm.at[pl.ds(bs(g), C)], dstv.at[p],
                                  semi.at[p]).start()

        def idx_wait(g, p):
            pltpu.make_async_copy(src_hbm.at[pl.ds(bs(g), C)], srcv.at[p],
                                  semi.at[p]).wait()
            pltpu.make_async_copy(dst_hbm.at[pl.ds(bs(g), C)], dstv.at[p],
                                  semi.at[p]).wait()

        def gat_start(g, p):
            pltpu.make_async_copy(q_hbm.at[dstv.at[p]], qb.at[p],
                                  semg.at[p]).start()
            pltpu.make_async_copy(k_hbm.at[srcv.at[p]], kb.at[p],
                                  semg.at[p]).start()

        def gat_wait(g, p):
            pltpu.make_async_copy(q_hbm.at[dstv.at[p]], qb.at[p],
                                  semg.at[p]).wait()
            pltpu.make_async_copy(k_hbm.at[srcv.at[p]], kb.at[p],
                                  semg.at[p]).wait()

        idx_start(0, 0)
        idx_wait(0, 0)
        gat_start(0, 0)
        idx_start(1, 1)

        def chunk(g, carry):
            p = lax.rem(g, 2)
            q = 1 - p
            gat_wait(g, p)

            @pl.when(g >= 1)
            def _():
                pltpu.make_async_copy(qb.at[q], qk_hbm.at[pl.ds(bs(g - 1), C)],
                                      semo.at[q]).wait()

            @pl.when(g < G - 1)
            def _():
                idx_wait(g + 1, q)
                gat_start(g + 1, q)

            def edge(i, c2):
                for h in range(HEADS):
                    sl = pl.ds(h * DH, DH)
                    qb[p, i, sl] = qb[p, i, sl] * kb[p, i, sl]
                return c2

            lax.fori_loop(0, C, edge, 0)
            pltpu.make_async_copy(qb.at[p], qk_hbm.at[pl.ds(bs(g), C)],
                                  semo.at[p]).start()

            @pl.when(g < G - 2)
            def _():
                idx_start(g + 2, p)

            return carry

        lax.fori_loop(0, G, chunk, 0)
        pltpu.make_async_copy(qb.at[lax.rem(G - 1, 2)],
                              qk_hbm.at[pl.ds(bs(G - 1), C)],
                              semo.at[lax.rem(G - 1, 2)]).wait()

    return pl.kernel(
        body,
        out_type=jax.ShapeDtypeStruct((E, HID), f32),
        mesh=mesh,
        scratch_types=[
            pltpu.VMEM((2, C), jnp.int32),
            pltpu.VMEM((2, C), jnp.int32),
            pltpu.VMEM((2, C, HID), f32),
            pltpu.VMEM((2, C, HID), f32),
            pltpu.SemaphoreType.DMA((2,)),
            pltpu.SemaphoreType.DMA((2,)),
            pltpu.SemaphoreType.DMA((2,)),
        ],
    )


@functools.lru_cache(maxsize=None)
def _build_sc_b():
    mesh = plsc.VectorSubcoreMesh(core_axis_name="c", subcore_axis_name="s")

    def body(v_hbm, wb_hbm, src_hbm, dst_hbm, ndn_hbm, ndd_hbm,
             srcv, dstv, vb, wbb, acc, semi, semg, semsc):
        cid = lax.axis_index("c")
        sid = lax.axis_index("s")
        wid = sid * NC + cid
        base_w = wid * EW
        z16 = jnp.zeros((16,), f32)

        def bs(g):
            return pl.multiple_of(base_w + g * C, 8)

        def zero_ndb():
            def zrow(r, carry):
                for j in range(HID // 16):
                    wbb[0, r, pl.ds(j * 16, 16)] = z16
                return carry

            lax.fori_loop(0, C, zrow, 0)

        def zero_acc():
            for j in range(RPT // C):
                pltpu.sync_copy(wbb.at[0],
                                acc.at[pl.ds(sid * RPT + j * C, C)])

        zero_ndb()
        zero_acc()
        plsc.subcore_barrier()

        def idx_start(g, p, with_src):
            if with_src:
                pltpu.make_async_copy(src_hbm.at[pl.ds(bs(g), C)],
                                      srcv.at[p], semi.at[p]).start()
            pltpu.make_async_copy(dst_hbm.at[pl.ds(bs(g), C)], dstv.at[p],
                                  semi.at[p]).start()

        def idx_wait(g, p, with_src):
            if with_src:
                pltpu.make_async_copy(src_hbm.at[pl.ds(bs(g), C)],
                                      srcv.at[p], semi.at[p]).wait()
            pltpu.make_async_copy(dst_hbm.at[pl.ds(bs(g), C)], dstv.at[p],
                                  semi.at[p]).wait()

        def gat_start(g, p, with_v):
            if with_v:
                pltpu.make_async_copy(v_hbm.at[srcv.at[p]], vb.at[p],
                                      semg.at[p]).start()
            pltpu.make_async_copy(wb_hbm.at[pl.ds(bs(g), C)], wbb.at[p],
                                  semg.at[p]).start()

        def gat_wait(g, p, with_v):
            if with_v:
                pltpu.make_async_copy(v_hbm.at[srcv.at[p]], vb.at[p],
                                      semg.at[p]).wait()
            pltpu.make_async_copy(wb_hbm.at[pl.ds(bs(g), C)], wbb.at[p],
                                  semg.at[p]).wait()

        def phase(with_v, out_hbm):
            idx_start(0, 0, with_v)
            idx_wait(0, 0, with_v)
            gat_start(0, 0, with_v)
            idx_start(1, 1, with_v)

            sbuf = vb if with_v else wbb

            def scat_wait(g, p):
                pltpu.make_async_copy(sbuf.at[p], acc.at[dstv.at[p]],
                                      semsc.at[p]).wait()

            def chunk(g, carry):
                p = lax.rem(g, 2)
                q = 1 - p
                gat_wait(g, p, with_v)

                @pl.when(g >= 1)
                def _():
                    scat_wait(g - 1, q)

                @pl.when(g < G - 1)
                def _():
                    idx_wait(g + 1, q, with_v)
                    gat_start(g + 1, q, with_v)

                if with_v:
                    def edge(i, c2):
                        for h in range(HEADS):
                            sl = pl.ds(h * DH, DH)
                            vb[p, i, sl] = vb[p, i, sl] * wbb[p, i, sl]
                        return c2

                    lax.fori_loop(0, C, edge, 0)
                pltpu.async_copy(sbuf.at[p], acc.at[dstv.at[p]],
                                 semsc.at[p], add=True)

                @pl.when(g < G - 2)
                def _():
                    idx_start(g + 2, p, with_v)

                return carry

            lax.fori_loop(0, G, chunk, 0)
            scat_wait(G - 1, lax.rem(G - 1, 2))
            plsc.subcore_barrier()
            pltpu.sync_copy(acc.at[pl.ds(sid * RPT, RPT)],
                            out_hbm.at[cid, pl.ds(sid * RPT, RPT)])

        phase(True, ndn_hbm)
        zero_ndb()
        zero_acc()
        plsc.subcore_barrier()
        phase(False, ndd_hbm)

    return pl.kernel(
        body,
        out_type=(jax.ShapeDtypeStruct((NC, NP, HID), f32),
                  jax.ShapeDtypeStruct((NC, NP, HID), f32)),
        mesh=mesh,
        scratch_types=[
            pltpu.VMEM((2, C), jnp.int32),
            pltpu.VMEM((2, C), jnp.int32),
            pltpu.VMEM((2, C, HID), f32),
            pltpu.VMEM((2, C, HID), f32),
            pltpu.VMEM_SHARED((NP, HID), f32),
            pltpu.SemaphoreType.DMA((2,)),
            pltpu.SemaphoreType.DMA((2,)),
            pltpu.SemaphoreType.DMA((2,)),
        ],
    )


def _node0_body(x, ne, wq, wk, wv, h_o, q_o, k_o, v_o):
    h = _mm(x[...], ne[...])
    h_o[...] = h
    q_o[...] = _mm(h, wq[...])
    k_o[...] = _mm(h, wk[...])
    v_o[...] = _mm(h, wv[...])


def _build_node0(interpret=False):
    return pl.pallas_call(
        _node0_body,
        out_shape=(jax.ShapeDtypeStruct((N, HID), f32),) * 4,
        interpret=interpret,
    )


def _attn_h2(h, ndn, ndd, wo, g1, b1, w1, bb1, w2, bb2, g2, b2):
    num = ndn[0, :N] + ndn[1, :N]
    den = ndd[0, :N] + ndd[1, :N]
    attn = num / (den + 1e-6)
    t = h + _mm(attn, wo)
    h1 = _bn(t, g1, b1)
    ff = _mm(jax.nn.relu(_mm(h1, w1) + bb1), w2) + bb2
    return _bn(h1 + ff, g2, b2)


def _node_mid_body(h, ndn, ndd, wo, g1, b1, w1, bb1, w2, bb2, g2, b2,
                   wq, wk, wv, h_o, q_o, k_o, v_o):
    h2 = _attn_h2(h[...], ndn[...], ndd[...], wo[...], g1[...], b1[...],
                  w1[...], bb1[...], w2[...], bb2[...], g2[...], b2[...])
    h_o[...] = h2
    q_o[...] = _mm(h2, wq[...])
    k_o[...] = _mm(h2, wk[...])
    v_o[...] = _mm(h2, wv[...])


def _build_node_mid(interpret=False):
    return pl.pallas_call(
        _node_mid_body,
        out_shape=(jax.ShapeDtypeStruct((N, HID), f32),) * 4,
        interpret=interpret,
    )


def _node_final_body(h, ndn, ndd, wo, g1, b1, w1, bb1, w2, bb2, g2, b2,
                     r0w, r0b, r1w, r1b, r2w, r2b, y_o):
    h2 = _attn_h2(h[...], ndn[...], ndd[...], wo[...], g1[...], b1[...],
                  w1[...], bb1[...], w2[...], bb2[...], g2[...], b2[...])
    y = jax.nn.relu(_mm(h2, r0w[...]) + r0b[...])
    y = jax.nn.relu(_mm(y, r1w[...]) + r1b[...])
    y_o[...] = _mm(y, r2w[...]) + r2b[...]


def _build_node_final(interpret=False):
    return pl.pallas_call(
        _node_final_body,
        out_shape=jax.ShapeDtypeStruct((N, 1), f32),
        interpret=interpret,
    )


BE = 4000
GE = E // BE


def _rowspec(w):
    return pl.BlockSpec((BE, w), lambda i: (i, 0))


def _cspec(shape):
    return pl.BlockSpec(shape, lambda i: (0,) * len(shape))


def _edge0_body(ea, eemb, we, e_o, ep_o):
    e0 = _mm(ea[...], eemb[...])
    e_o[...] = e0
    ep_o[...] = _mm(e0, we[...])


def _build_edge0(interpret=False):
    return pl.pallas_call(
        _edge0_body,
        grid=(GE,),
        in_specs=[_rowspec(16), _cspec((16, HID)), _cspec((HID, HID))],
        out_specs=(_rowspec(HID), _rowspec(HID)),
        out_shape=(jax.ShapeDtypeStruct((E, HID), f32),) * 2,
        interpret=interpret,
    )


def _accum_stats(i, stats_o, t):
    s0 = jnp.sum(t, axis=0, keepdims=True)
    s1 = jnp.sum(t * t, axis=0, keepdims=True)
    blk = jnp.concatenate([s0, s1, jnp.zeros((6, HID), f32)], axis=0)

    @pl.when(i == 0)
    def _():
        stats_o[...] = blk

    @pl.when(i > 0)
    def _():
        stats_o[...] = stats_o[...] + blk


def _wb_rows(sc, denbt, denb):
    s8 = _mm(sc, denbt)
    w = jnp.exp(jnp.minimum(jnp.maximum(s8, -5.0), 5.0))
    return _mm(w, denb)


def _edge_p1_body(e, qk, ep, woe, denbt, denb, t_o, stats_o, wb_o):
    i = pl.program_id(0)
    sc = qk[...] * ep[...] * 0.25
    t = e[...] + _mm(sc, woe[...])
    t_o[...] = t
    _accum_stats(i, stats_o, t)
    wb_o[...] = _wb_rows(sc, denbt[...], denb[...])


def _build_edge_p1(interpret=False):
    return pl.pallas_call(
        _edge_p1_body,
        grid=(GE,),
        in_specs=[_rowspec(HID), _rowspec(HID), _rowspec(HID),
                  _cspec((HID, HID)),
                  _cspec((HID, HEADS)), _cspec((HEADS, HID))],
        out_specs=(_rowspec(HID), _cspec((8, HID)), _rowspec(HID)),
        out_shape=(jax.ShapeDtypeStruct((E, HID), f32),
                   jax.ShapeDtypeStruct((8, HID), f32),
                   jax.ShapeDtypeStruct((E, HID), f32)),
        interpret=interpret,
    )


def _mid_body(qk, ep, denbt, denb, wb_o):
    wb_o[...] = _wb_rows(qk[...] * ep[...] * 0.25, denbt[...], denb[...])


def _build_mid(interpret=False):
    return pl.pallas_call(
        _mid_body,
        grid=(GE,),
        in_specs=[_rowspec(HID), _rowspec(HID), _cspec((HID, HEADS)),
                  _cspec((HEADS, HID))],
        out_specs=_rowspec(HID),
        out_shape=jax.ShapeDtypeStruct((E, HID), f32),
        interpret=interpret,
    )


def _apply_stats(t, stats, g, b):
    mean = stats[0:1, :] * (1.0 / E)
    var = stats[1:2, :] * (1.0 / E) - mean * mean
    return (t - mean) * lax.rsqrt(var + 1e-5) * g + b


def _edge_p2_body(t, stats, ge1, be1, we1, bbe1, we2, bbe2, u_o, stats_o):
    i = pl.program_id(0)
    e1 = _apply_stats(t[...], stats[...], ge1[...], be1[...])
    u = e1 + _mm(jax.nn.relu(_mm(e1, we1[...]) + bbe1[...]), we2[...]) + bbe2[...]
    u_o[...] = u
    _accum_stats(i, stats_o, u)


def _build_edge_p2(interpret=False):
    return pl.pallas_call(
        _edge_p2_body,
        grid=(GE,),
        in_specs=[_rowspec(HID), _cspec((8, HID)), _cspec((1, HID)),
                  _cspec((1, HID)), _cspec((HID, 2 * HID)), _cspec((1, 2 * HID)),
                  _cspec((2 * HID, HID)), _cspec((1, HID))],
        out_specs=(_rowspec(HID), _cspec((8, HID))),
        out_shape=(jax.ShapeDtypeStruct((E, HID), f32),
                   jax.ShapeDtypeStruct((8, HID), f32)),
        interpret=interpret,
    )


def _edge_p3_body(u, stats, ge2, be2, wen, e_o, ep_o):
    e2 = _apply_stats(u[...], stats[...], ge2[...], be2[...])
    e_o[...] = e2
    ep_o[...] = _mm(e2, wen[...])


def _edge_p3_noe_body(u, stats, ge2, be2, wen, ep_o):
    e2 = _apply_stats(u[...], stats[...], ge2[...], be2[...])
    ep_o[...] = _mm(e2, wen[...])


def _build_edge_p3(emit_e, interpret=False):
    specs = [_rowspec(HID), _cspec((8, HID)), _cspec((1, HID)),
             _cspec((1, HID)), _cspec((HID, HID))]
    if emit_e:
        return pl.pallas_call(
            _edge_p3_body,
            grid=(GE,),
            in_specs=specs,
            out_specs=(_rowspec(HID), _rowspec(HID)),
            out_shape=(jax.ShapeDtypeStruct((E, HID), f32),) * 2,
            interpret=interpret,
        )
    return pl.pallas_call(
        _edge_p3_noe_body,
        grid=(GE,),
        in_specs=specs,
        out_specs=(_rowspec(HID),),
        out_shape=(jax.ShapeDtypeStruct((E, HID), f32),),
        interpret=interpret,
    )


_node0 = _build_node0()
_node_mid = _build_node_mid()
_node_final = _build_node_final()
_edge0 = _build_edge0()
_edge_p1 = _build_edge_p1()
_mid = _build_mid()
_edge_p2 = _build_edge_p2()
_edge_p3 = _build_edge_p3(True)
_edge_p3_noe = _build_edge_p3(False)


def _row(v):
    return v.reshape(1, -1)


def kernel(x, edge_attr, params, edge_index):
    src = edge_index[0]
    dst = edge_index[1]
    denb = jnp.asarray(_DENB_NP)
    denbt = jnp.asarray(_DENB_NP.T)
    p0 = params["layers"][0]
    h, Q, K, V = _node0(x, params["node_emb"], p0["Wq"], p0["Wk"], p0["Wv"])
    e, Ep = _edge0(edge_attr, params["edge_emb"], p0["We"])
    for i in range(LAYERS):
        li = params["layers"][i]
        node_args = (li["Wo"], _row(li["g1"]), _row(li["b1"]),
                     li["W1"], _row(li["bb1"]), li["W2"], _row(li["bb2"]),
                     _row(li["g2"]), _row(li["b2"]))
        qk = _build_sc_a()(Q, K, src, dst)
        if i < LAYERS - 1:
            t, st, wb = _edge_p1(e, qk, Ep, li["Woe"], denbt, denb)
        else:
            wb = _mid(qk, Ep, denbt, denb)
        ndn, ndd = _build_sc_b()(V, wb, src, dst)
        if i < LAYERS - 1:
            nl = params["layers"][i + 1]
            h, Q, K, V = _node_mid(h, ndn, ndd, *node_args,
                                   nl["Wq"], nl["Wk"], nl["Wv"])
            u, st2 = _edge_p2(t, st, _row(li["ge1"]), _row(li["be1"]),
                              li["We1"], _row(li["bbe1"]),
                              li["We2"], _row(li["bbe2"]))
            if i < LAYERS - 2:
                e, Ep = _edge_p3(u, st2, _row(li["ge2"]), _row(li["be2"]),
                                 nl["We"])
            else:
                (Ep,) = _edge_p3_noe(u, st2, _row(li["ge2"]), _row(li["be2"]),
                                     nl["We"])
        else:
            y = _node_final(h, ndn, ndd, *node_args,
                            params["r0W"], _row(params["r0b"]),
                            params["r1W"], _row(params["r1b"]),
                            params["r2W"], _row(params["r2b"]))
    return y
